# baseline (speedup 1.0000x reference)
import functools

import jax
import jax.numpy as jnp
from jax import lax
from jax.experimental import pallas as pl
from jax.experimental.pallas import tpu as pltpu
from jax.experimental.pallas import tpu_sc as plsc

NC = 2
NS = 16
NW = NC * NS
LANES = 16
GSUB = 128
BW = 512


def _tc_rowpack_table(table):
    q, d = table.shape
    t_t = table.T
    grid = (q + BW - 1) // BW
    rpb = BW * d // 128

    def body(x_ref, y_ref):
        x = x_ref[...]
        y_ref[...] = jnp.concatenate(
            [x[:, 128 * a:128 * (a + 1)].T for a in range(4)], axis=1)

    out = pl.pallas_call(
        body,
        grid=(grid,),
        in_specs=[pl.BlockSpec((d, BW), lambda i: (0, i))],
        out_specs=pl.BlockSpec((rpb, 128), lambda i: (i, 0)),
        out_shape=jax.ShapeDtypeStruct((grid * rpb, 128), jnp.float32),
    )(t_t)
    return out.reshape(grid * BW, d)


def _sc_unified_gather(idx2d, table_rm, n_feat, batch, d):
    chunk = batch // NW

    mesh = plsc.VectorSubcoreMesh(core_axis_name="c", subcore_axis_name="s")

    @functools.partial(
        pl.kernel,
        mesh=mesh,
        out_type=[jax.ShapeDtypeStruct((batch, d), jnp.float32)
                  for _ in range(n_feat)],
        compiler_params=pltpu.CompilerParams(use_tc_tiling_on_sc=False),
        scratch_types=[
            pltpu.VMEM((chunk,), jnp.int32),
            pltpu.VMEM((chunk,), jnp.int32),
            pltpu.VMEM((chunk, d), jnp.float32),
            pltpu.VMEM((chunk, d), jnp.float32),
        ] + [pltpu.SemaphoreType.DMA] * 6,
    )
    def sc_kernel(idx_hbm, table_hbm, *rest):
        outs = rest[:n_feat]
        idxb = rest[n_feat:n_feat + 2]
        rows = rest[n_feat + 2:n_feat + 4]
        semi = rest[n_feat + 4:n_feat + 6]
        semg = rest[n_feat + 6:n_feat + 8]
        semw = rest[n_feat + 8:n_feat + 10]
        wid = lax.axis_index("s") * NC + lax.axis_index("c")
        base = wid * chunk
        q = 1000000

        def issue_idx_load(f, b):
            return pltpu.async_copy(
                idx_hbm.at[f, pl.ds(base, chunk)], idxb[b], semi[b])

        def hash_chunk(f, b):
            salt = f * 7919

            @pl.loop(0, chunk, step=LANES)
            def _(t):
                sl = pl.ds(t, LANES)
                r = (idxb[b][sl] * 31 + salt) % q
                idxb[b][sl] = ((r >> 9) << 9) + ((r & 127) << 2) + ((r >> 7) & 3)

        def issue_gathers(f, b):
            return [
                pltpu.async_copy(
                    table_hbm.at[idxb[b].at[pl.ds(g * GSUB, GSUB)]],
                    rows[b].at[pl.ds(g * GSUB, GSUB)],
                    semg[b],
                )
                for g in range(chunk // GSUB)
            ]

        def issue_write(f, b):
            return pltpu.async_copy(
                rows[b], outs[f].at[pl.ds(base, chunk)], semw[b])

        ih = [None] * (n_feat + 2)
        gh = [None] * (n_feat + 1)
        wh = [None] * (n_feat + 1)
        ih[0] = issue_idx_load(0, 0)
        ih[1] = issue_idx_load(1, 1)
        ih[0].wait()
        hash_chunk(0, 0)
        gh[0] = issue_gathers(0, 0)

        for f in range(n_feat):
            b = f % 2
            if f + 1 < n_feat:
                ih[f + 1].wait()
                hash_chunk(f + 1, 1 - b)
                gh[f + 1] = issue_gathers(f + 1, 1 - b)
            for h in gh[f]:
                h.wait()
            if wh[f - 1] is not None:
                wh[f - 1].wait()
            wh[f] = issue_write(f, b)
            if f + 2 < n_feat:
                ih[f + 2] = issue_idx_load(f + 2, b)
        wh[n_feat - 1].wait()

    return sc_kernel(idx2d, table_rm)


def kernel(inputs, table):
    n_feat, batch, _ = inputs.shape
    d = table.shape[1]
    idx2d = inputs.reshape(n_feat, batch)
    table_rm = _tc_rowpack_table(table)
    return tuple(_sc_unified_gather(idx2d, table_rm, n_feat, batch, d))


# speedup vs baseline: 1.5512x; 1.5512x over previous
import functools

import jax
import jax.numpy as jnp
from jax import lax
from jax.experimental import pallas as pl
from jax.experimental.pallas import tpu as pltpu
from jax.experimental.pallas import tpu_sc as plsc

NC = 2
NS = 16
NW = NC * NS
LANES = 16
GSUB = 128
BW = 2048


def _tc_rowpack_table(table):
    q, d = table.shape
    t_t = table.T
    grid = (q + BW - 1) // BW
    rpb = BW * d // 128
    sub = BW // 4

    def body(x_ref, y_ref):
        eye = jnp.eye(d, dtype=jnp.float32)
        for a in range(4):
            x = x_ref[:, a * sub:(a + 1) * sub]
            xt = jax.lax.dot_general(
                x, eye, (((0,), (0,)), ((), ())),
                precision=jax.lax.Precision.HIGHEST,
                preferred_element_type=jnp.float32)
            y_ref[:, a * d:(a + 1) * d] = xt

    out = pl.pallas_call(
        body,
        grid=(grid,),
        in_specs=[pl.BlockSpec((d, BW), lambda i: (0, i))],
        out_specs=pl.BlockSpec((rpb, 128), lambda i: (i, 0)),
        out_shape=jax.ShapeDtypeStruct((grid * rpb, 128), jnp.float32),
        compiler_params=pltpu.CompilerParams(
            dimension_semantics=("arbitrary",)),
    )(t_t)
    return out.reshape(grid * BW, d)


def _sc_unified_gather(idx2d, table_rm, n_feat, batch, d):
    chunk = batch // NW

    mesh = plsc.VectorSubcoreMesh(core_axis_name="c", subcore_axis_name="s")

    @functools.partial(
        pl.kernel,
        mesh=mesh,
        out_type=[jax.ShapeDtypeStruct((batch, d), jnp.float32)
                  for _ in range(n_feat)],
        compiler_params=pltpu.CompilerParams(use_tc_tiling_on_sc=False),
        scratch_types=[
            pltpu.VMEM((chunk,), jnp.int32),
            pltpu.VMEM((chunk,), jnp.int32),
            pltpu.VMEM((chunk, d), jnp.float32),
            pltpu.VMEM((chunk, d), jnp.float32),
        ] + [pltpu.SemaphoreType.DMA] * 6,
    )
    def sc_kernel(idx_hbm, table_hbm, *rest):
        outs = rest[:n_feat]
        idxb = rest[n_feat:n_feat + 2]
        rows = rest[n_feat + 2:n_feat + 4]
        semi = rest[n_feat + 4:n_feat + 6]
        semg = rest[n_feat + 6:n_feat + 8]
        semw = rest[n_feat + 8:n_feat + 10]
        wid = lax.axis_index("s") * NC + lax.axis_index("c")
        base = wid * chunk
        q = 1000000

        def issue_idx_load(f, b):
            return pltpu.async_copy(
                idx_hbm.at[f, pl.ds(base, chunk)], idxb[b], semi[b])

        def hash_chunk(f, b):
            salt = f * 7919

            @pl.loop(0, chunk, step=LANES)
            def _(t):
                sl = pl.ds(t, LANES)
                r = (idxb[b][sl] * 31 + salt) % q
                idxb[b][sl] = (((r >> 11) << 11) + ((r & 511) << 2)
                               + ((r >> 9) & 3))

        def issue_gathers(f, b):
            return [
                pltpu.async_copy(
                    table_hbm.at[idxb[b].at[pl.ds(g * GSUB, GSUB)]],
                    rows[b].at[pl.ds(g * GSUB, GSUB)],
                    semg[b],
                )
                for g in range(chunk // GSUB)
            ]

        def issue_write(f, b):
            return pltpu.async_copy(
                rows[b], outs[f].at[pl.ds(base, chunk)], semw[b])

        ih = [None] * (n_feat + 2)
        gh = [None] * (n_feat + 1)
        wh = [None] * (n_feat + 1)
        ih[0] = issue_idx_load(0, 0)
        ih[1] = issue_idx_load(1, 1)
        ih[0].wait()
        hash_chunk(0, 0)
        gh[0] = issue_gathers(0, 0)

        for f in range(n_feat):
            b = f % 2
            if f + 1 < n_feat:
                ih[f + 1].wait()
                hash_chunk(f + 1, 1 - b)
                gh[f + 1] = issue_gathers(f + 1, 1 - b)
            for h in gh[f]:
                h.wait()
            if wh[f - 1] is not None:
                wh[f - 1].wait()
            wh[f] = issue_write(f, b)
            if f + 2 < n_feat:
                ih[f + 2] = issue_idx_load(f + 2, b)
        wh[n_feat - 1].wait()

    return sc_kernel(idx2d, table_rm)


def kernel(inputs, table):
    n_feat, batch, _ = inputs.shape
    d = table.shape[1]
    idx2d = inputs.reshape(n_feat, batch)
    table_rm = _tc_rowpack_table(table)
    return tuple(_sc_unified_gather(idx2d, table_rm, n_feat, batch, d))


# speedup vs baseline: 1.7785x; 1.1465x over previous
import functools

import jax
import jax.numpy as jnp
from jax import lax
from jax.experimental import pallas as pl
from jax.experimental.pallas import tpu as pltpu
from jax.experimental.pallas import tpu_sc as plsc

NC = 2
NS = 16
NW = NC * NS
LANES = 16
GSUB = 128


def _sc_unified_gather(idx2d, table_rm, n_feat, batch, d):
    chunk = batch // NW

    mesh = plsc.VectorSubcoreMesh(core_axis_name="c", subcore_axis_name="s")

    @functools.partial(
        pl.kernel,
        mesh=mesh,
        out_type=[jax.ShapeDtypeStruct((batch, d), jnp.float32)
                  for _ in range(n_feat)],
        compiler_params=pltpu.CompilerParams(use_tc_tiling_on_sc=False),
        scratch_types=[
            pltpu.VMEM((chunk,), jnp.int32),
            pltpu.VMEM((chunk,), jnp.int32),
            pltpu.VMEM((chunk, d), jnp.float32),
            pltpu.VMEM((chunk, d), jnp.float32),
        ] + [pltpu.SemaphoreType.DMA] * 6,
    )
    def sc_kernel(idx_hbm, table_hbm, *rest):
        outs = rest[:n_feat]
        idxb = rest[n_feat:n_feat + 2]
        rows = rest[n_feat + 2:n_feat + 4]
        semi = rest[n_feat + 4:n_feat + 6]
        semg = rest[n_feat + 6:n_feat + 8]
        semw = rest[n_feat + 8:n_feat + 10]
        wid = lax.axis_index("s") * NC + lax.axis_index("c")
        base = wid * chunk
        q = 1000000

        def issue_idx_load(f, b):
            return pltpu.async_copy(
                idx_hbm.at[f, pl.ds(base, chunk)], idxb[b], semi[b])

        def hash_chunk(f, b):
            salt = f * 7919

            @pl.loop(0, chunk, step=LANES)
            def _(t):
                sl = pl.ds(t, LANES)
                idxb[b][sl] = (idxb[b][sl] * 31 + salt) % q

        def issue_gathers(f, b):
            return [
                pltpu.async_copy(
                    table_hbm.at[idxb[b].at[pl.ds(g * GSUB, GSUB)]],
                    rows[b].at[pl.ds(g * GSUB, GSUB)],
                    semg[b],
                )
                for g in range(chunk // GSUB)
            ]

        def issue_write(f, b):
            return pltpu.async_copy(
                rows[b], outs[f].at[pl.ds(base, chunk)], semw[b])

        ih = [None] * (n_feat + 2)
        gh = [None] * (n_feat + 1)
        wh = [None] * (n_feat + 1)
        ih[0] = issue_idx_load(0, 0)
        ih[1] = issue_idx_load(1, 1)
        ih[0].wait()
        hash_chunk(0, 0)
        gh[0] = issue_gathers(0, 0)

        for f in range(n_feat):
            b = f % 2
            if f + 1 < n_feat:
                ih[f + 1].wait()
                hash_chunk(f + 1, 1 - b)
                gh[f + 1] = issue_gathers(f + 1, 1 - b)
            for h in gh[f]:
                h.wait()
            if wh[f - 1] is not None:
                wh[f - 1].wait()
            wh[f] = issue_write(f, b)
            if f + 2 < n_feat:
                ih[f + 2] = issue_idx_load(f + 2, b)
        wh[n_feat - 1].wait()

    return sc_kernel(idx2d, table_rm)


def kernel(inputs, table):
    n_feat, batch, _ = inputs.shape
    d = table.shape[1]
    idx2d = inputs.reshape(n_feat, batch)
    return tuple(_sc_unified_gather(idx2d, table, n_feat, batch, d))
